# Initial kernel scaffold; baseline (speedup 1.0000x reference)
import jax
import jax.numpy as jnp
from jax.experimental import pallas as pl


def kernel(outA, outB, matchA, matchB, nonMatchA, nonMatchB, hardNegative):
    raise NotImplementedError("write your pallas kernel here")


# speedup vs baseline: 2.4346x; 2.4346x over previous
import functools

import jax
import jax.numpy as jnp
from jax import lax
from jax.experimental import pallas as pl
from jax.experimental.pallas import tpu as pltpu
from jax.experimental.pallas import tpu_sc as plsc

B = 2
N_PIX = 147456
D = 128
N_MATCH = 5000
N_NONMATCH = 20000
MARGIN = 0.5
NONMATCH_W = 1.0

NC = 2
NS = 16
NW = NC * NS
H = 2
HW = D // H
L = 16


def _round8(x):
    return (x + 7) // 8 * 8


M_PER_W = B * N_MATCH * H // NW
NM_PER_W = B * N_NONMATCH * H // NW
M_PAD = _round8(M_PER_W)
NM_PAD = _round8(NM_PER_W)
IDX_TOT = 2 * (M_PAD + NM_PAD)
OFF_AM = 0
OFF_BM = M_PAD
OFF_ANM = 2 * M_PAD
OFF_BNM = 2 * M_PAD + NM_PAD

T = 128


def _tiles(total):
    out = []
    s = 0
    while s < total:
        out.append((s, min(T, total - s)))
        s += T
    return out


M_TILES = _tiles(M_PER_W)
NM_TILES = _tiles(NM_PER_W)

_mesh = plsc.VectorSubcoreMesh(core_axis_name="c", subcore_axis_name="s")


@functools.partial(
    pl.kernel,
    out_type=jax.ShapeDtypeStruct((NW, 2 * L), jnp.float32),
    mesh=_mesh,
    scratch_types=[
        pltpu.VMEM((IDX_TOT,), jnp.int32),
        pltpu.VMEM((T, HW), jnp.float32),
        pltpu.VMEM((T, HW), jnp.float32),
        pltpu.VMEM((2 * L,), jnp.float32),
        pltpu.SemaphoreType.DMA,
    ],
    compiler_params=pltpu.CompilerParams(use_tc_tiling_on_sc=False),
)
def _sc_loss(tableA, tableB, idx_all, out, idx_v, bufA, bufB, stage, sem):
    wid = lax.axis_index("s") * NC + lax.axis_index("c")
    pltpu.sync_copy(idx_all.at[wid], idx_v)

    def gather(offA, offB, start, size):
        cpA = pltpu.async_copy(
            tableA.at[idx_v.at[pl.ds(offA + start, size)]],
            bufA.at[pl.ds(0, size)], sem)
        cpB = pltpu.async_copy(
            tableB.at[idx_v.at[pl.ds(offB + start, size)]],
            bufB.at[pl.ds(0, size)], sem)
        cpA.wait()
        cpB.wait()

    zero = jnp.zeros((L,), jnp.float32)

    def match_body(r, acc):
        for j in range(HW // L):
            a = bufA[r, pl.ds(j * L, L)]
            b = bufB[r, pl.ds(j * L, L)]
            d = a - b
            acc = acc + d * d
        return acc

    def nonmatch_body(r, acc):
        for j in range(HW // L):
            a = bufA[r, pl.ds(j * L, L)]
            b = bufB[r, pl.ds(j * L, L)]
            d = a - b
            acc = acc + jnp.maximum(MARGIN - d * d, zero)
        return acc

    acc_m = zero
    for (start, size) in M_TILES:
        gather(OFF_AM, OFF_BM, start, size)
        acc_m = lax.fori_loop(0, size, match_body, acc_m)

    acc_nm = zero
    for (start, size) in NM_TILES:
        gather(OFF_ANM, OFF_BNM, start, size)
        acc_nm = lax.fori_loop(0, size, nonmatch_body, acc_nm)

    stage[pl.ds(0, L)] = acc_m
    stage[pl.ds(L, L)] = acc_nm
    pltpu.sync_copy(stage, out.at[wid])


def _half_indices(idx, n_valid, n_pad):
    base = (idx.astype(jnp.int32)
            + (jnp.arange(B, dtype=jnp.int32) * N_PIX)[:, None]) * H
    inter = jnp.stack([base, base + 1], axis=-1).reshape(NW, n_valid)
    return jnp.pad(inter, ((0, 0), (0, n_pad - n_valid)))


def kernel(outA, outB, matchA, matchB, nonMatchA, nonMatchB, hardNegative):
    tableA = outA.reshape(B * N_PIX * H, HW)
    tableB = outB.reshape(B * N_PIX * H, HW)
    idx_all = jnp.concatenate([
        _half_indices(matchA, M_PER_W, M_PAD),
        _half_indices(matchB, M_PER_W, M_PAD),
        _half_indices(nonMatchA, NM_PER_W, NM_PAD),
        _half_indices(nonMatchB, NM_PER_W, NM_PAD),
    ], axis=1)

    parts = _sc_loss(tableA, tableB, idx_all)
    matchLossSum = parts[:, :L].sum() / N_MATCH
    nonMatchLossSum = NONMATCH_W * parts[:, L:].sum() / N_NONMATCH
    contrastiveLossSum = matchLossSum + nonMatchLossSum
    return (contrastiveLossSum, matchLossSum, nonMatchLossSum)


# speedup vs baseline: 5.9696x; 2.4520x over previous
import functools

import jax
import jax.numpy as jnp
from jax import lax
from jax.experimental import pallas as pl
from jax.experimental.pallas import tpu as pltpu
from jax.experimental.pallas import tpu_sc as plsc

B = 2
N_PIX = 147456
D = 128
N_MATCH = 5000
N_NONMATCH = 20000
MARGIN = 0.5
NONMATCH_W = 1.0

NC = 2
NS = 16
NW = NC * NS
L = 16

M_TOT = B * N_MATCH
NM_TOT = B * N_NONMATCH
M_PER_W = -(-M_TOT // NW)
M_PAD = 320
NM_PER_W = NM_TOT // NW
NM_PAD = 1256

T = 128


def _tiles(total):
    out, s = [], 0
    while s < total:
        out.append((s, min(T, total - s)))
        s += T
    return out


M_TILES = _tiles(M_PER_W)
NM_TILES = _tiles(NM_PER_W)

_mesh = plsc.VectorSubcoreMesh(core_axis_name="c", subcore_axis_name="s")


@functools.partial(
    pl.kernel,
    out_type=jax.ShapeDtypeStruct((NW, 2 * L), jnp.float32),
    mesh=_mesh,
    scratch_types=[
        pltpu.VMEM((M_PAD,), jnp.int32),
        pltpu.VMEM((M_PAD,), jnp.int32),
        pltpu.VMEM((NM_PAD,), jnp.int32),
        pltpu.VMEM((NM_PAD,), jnp.int32),
        pltpu.VMEM((T, D), jnp.float32),
        pltpu.VMEM((T, D), jnp.float32),
        pltpu.VMEM((T, D), jnp.float32),
        pltpu.VMEM((T, D), jnp.float32),
        pltpu.VMEM((2 * L,), jnp.float32),
        pltpu.SemaphoreType.DMA,
        pltpu.SemaphoreType.DMA,
        pltpu.SemaphoreType.DMA,
    ],
)
def _sc_loss(tableA, tableB, idxAm, idxBm, idxAnm, idxBnm, out,
             iAm_v, iBm_v, iAnm_v, iBnm_v,
             bufA0, bufB0, bufA1, bufB1, stage, sem0, sem1, sem_i):
    wid = lax.axis_index("s") * NC + lax.axis_index("c")

    cps = [pltpu.async_copy(idxAm.at[pl.ds(wid * M_PAD, M_PAD)], iAm_v, sem_i),
           pltpu.async_copy(idxBm.at[pl.ds(wid * M_PAD, M_PAD)], iBm_v, sem_i),
           pltpu.async_copy(idxAnm.at[pl.ds(wid * NM_PAD, NM_PAD)], iAnm_v, sem_i),
           pltpu.async_copy(idxBnm.at[pl.ds(wid * NM_PAD, NM_PAD)], iBnm_v, sem_i)]
    for cp in cps:
        cp.wait()

    m_valid = jnp.minimum(M_PER_W, M_TOT - wid * M_PER_W)

    bufs = [(bufA0, bufB0, sem0), (bufA1, bufB1, sem1)]
    tiles = [(True, s, z) for (s, z) in M_TILES] + \
            [(False, s, z) for (s, z) in NM_TILES]

    def issue(i):
        is_m, start, size = tiles[i]
        bA, bB, sem = bufs[i % 2]
        ia = (iAm_v if is_m else iAnm_v).at[pl.ds(start, size)]
        ib = (iBm_v if is_m else iBnm_v).at[pl.ds(start, size)]
        cpA = pltpu.async_copy(tableA.at[ia], bA.at[pl.ds(0, size)], sem)
        cpB = pltpu.async_copy(tableB.at[ib], bB.at[pl.ds(0, size)], sem)
        return cpA, cpB

    zero = jnp.zeros((L,), jnp.float32)

    def make_body(bA, bB, is_m):
        def body(r, acc):
            for j in range(D // L):
                a = bA[r, pl.ds(j * L, L)]
                b = bB[r, pl.ds(j * L, L)]
                d = a - b
                if is_m:
                    acc = acc + d * d
                else:
                    acc = acc + jnp.maximum(MARGIN - d * d, zero)
            return acc
        return body

    acc_m = zero
    acc_nm = zero
    inflight = issue(0)
    for i, (is_m, start, size) in enumerate(tiles):
        cpA, cpB = inflight
        if i + 1 < len(tiles):
            inflight = issue(i + 1)
        cpA.wait()
        cpB.wait()
        bA, bB, _ = bufs[i % 2]
        if is_m:
            n = jnp.clip(m_valid - start, 0, size)
            acc_m = lax.fori_loop(0, n, make_body(bA, bB, True), acc_m)
        else:
            acc_nm = lax.fori_loop(0, size, make_body(bA, bB, False), acc_nm)

    stage[pl.ds(0, L)] = acc_m
    stage[pl.ds(L, L)] = acc_nm
    pltpu.sync_copy(stage, out.at[wid])


def _prep(idx, per_w, per_pad):
    biased = idx.astype(jnp.int32) + (jnp.arange(B, dtype=jnp.int32) * N_PIX)[:, None]
    flat = biased.reshape(-1)
    total = flat.shape[0]
    flat = jnp.pad(flat, (0, NW * per_w - total))
    return jnp.pad(flat.reshape(NW, per_w), ((0, 0), (0, per_pad - per_w))).reshape(-1)


def kernel(outA, outB, matchA, matchB, nonMatchA, nonMatchB, hardNegative):
    tableA = outA.reshape(B * N_PIX, D)
    tableB = outB.reshape(B * N_PIX, D)
    parts = _sc_loss(
        tableA, tableB,
        _prep(matchA, M_PER_W, M_PAD),
        _prep(matchB, M_PER_W, M_PAD),
        _prep(nonMatchA, NM_PER_W, NM_PAD),
        _prep(nonMatchB, NM_PER_W, NM_PAD),
    )
    matchLossSum = parts[:, :L].sum() / N_MATCH
    nonMatchLossSum = NONMATCH_W * parts[:, L:].sum() / N_NONMATCH
    contrastiveLossSum = matchLossSum + nonMatchLossSum
    return (contrastiveLossSum, matchLossSum, nonMatchLossSum)


# speedup vs baseline: 5.9873x; 1.0030x over previous
import functools

import jax
import jax.numpy as jnp
from jax import lax
from jax.experimental import pallas as pl
from jax.experimental.pallas import tpu as pltpu
from jax.experimental.pallas import tpu_sc as plsc

B = 2
N_PIX = 147456
D = 128
N_MATCH = 5000
N_NONMATCH = 20000
MARGIN = 0.5
NONMATCH_W = 1.0

NC = 2
NS = 16
NW = NC * NS
L = 16

M_TOT = B * N_MATCH
NM_TOT = B * N_NONMATCH
M_PER_W = -(-M_TOT // NW)
M_PAD = 320
NM_PER_W = NM_TOT // NW
NM_PAD = 1256

T = 128


def _tiles(total):
    out, s = [], 0
    while s < total:
        out.append((s, min(T, total - s)))
        s += T
    return out


M_TILES = _tiles(M_PER_W)
NM_TILES = _tiles(NM_PER_W)

_mesh = plsc.VectorSubcoreMesh(core_axis_name="c", subcore_axis_name="s")


@functools.partial(
    pl.kernel,
    out_type=jax.ShapeDtypeStruct((NW, 2 * L), jnp.float32),
    mesh=_mesh,
    scratch_types=[
        pltpu.VMEM((M_PAD,), jnp.int32),
        pltpu.VMEM((M_PAD,), jnp.int32),
        pltpu.VMEM((NM_PAD,), jnp.int32),
        pltpu.VMEM((NM_PAD,), jnp.int32),
        pltpu.VMEM((T, D), jnp.float32),
        pltpu.VMEM((T, D), jnp.float32),
        pltpu.VMEM((T, D), jnp.float32),
        pltpu.VMEM((T, D), jnp.float32),
        pltpu.VMEM((2 * L,), jnp.float32),
        pltpu.SemaphoreType.DMA,
        pltpu.SemaphoreType.DMA,
        pltpu.SemaphoreType.DMA,
    ],
)
def _sc_loss(tableA, tableB, idxAm, idxBm, idxAnm, idxBnm, out,
             iAm_v, iBm_v, iAnm_v, iBnm_v,
             bufA0, bufB0, bufA1, bufB1, stage, sem0, sem1, sem_i):
    wid = lax.axis_index("s") * NC + lax.axis_index("c")

    cps = [pltpu.async_copy(idxAm.at[pl.ds(wid * M_PAD, M_PAD)], iAm_v, sem_i),
           pltpu.async_copy(idxBm.at[pl.ds(wid * M_PAD, M_PAD)], iBm_v, sem_i),
           pltpu.async_copy(idxAnm.at[pl.ds(wid * NM_PAD, NM_PAD)], iAnm_v, sem_i),
           pltpu.async_copy(idxBnm.at[pl.ds(wid * NM_PAD, NM_PAD)], iBnm_v, sem_i)]
    for cp in cps:
        cp.wait()

    m_valid = jnp.minimum(M_PER_W, M_TOT - wid * M_PER_W)

    bufs = [(bufA0, bufB0, sem0), (bufA1, bufB1, sem1)]
    tiles = [(True, s, z) for (s, z) in M_TILES] + \
            [(False, s, z) for (s, z) in NM_TILES]

    def issue(i):
        is_m, start, size = tiles[i]
        bA, bB, sem = bufs[i % 2]
        ia = (iAm_v if is_m else iAnm_v).at[pl.ds(start, size)]
        ib = (iBm_v if is_m else iBnm_v).at[pl.ds(start, size)]
        cpA = pltpu.async_copy(tableA.at[ia], bA.at[pl.ds(0, size)], sem)
        cpB = pltpu.async_copy(tableB.at[ib], bB.at[pl.ds(0, size)], sem)
        return cpA, cpB

    zero = jnp.zeros((L,), jnp.float32)

    def run_tile(bA, bB, n, accs, is_m):
        def body(r, acc):
            acc = list(acc)
            for j in range(D // L):
                a = bA[r, pl.ds(j * L, L)]
                b = bB[r, pl.ds(j * L, L)]
                d = a - b
                if is_m:
                    acc[j % 4] = acc[j % 4] + d * d
                else:
                    acc[j % 4] = acc[j % 4] + jnp.maximum(MARGIN - d * d, zero)
            return tuple(acc)
        return plsc.parallel_loop(0, n, 1, unroll=2, carry=accs)(body)

    acc_m = (zero, zero, zero, zero)
    acc_nm = (zero, zero, zero, zero)
    inflight = issue(0)
    for i, (is_m, start, size) in enumerate(tiles):
        cpA, cpB = inflight
        if i + 1 < len(tiles):
            inflight = issue(i + 1)
        cpA.wait()
        cpB.wait()
        bA, bB, _ = bufs[i % 2]
        if is_m:
            n = jnp.clip(m_valid - start, 0, size)
            acc_m = run_tile(bA, bB, n, acc_m, True)
        else:
            acc_nm = run_tile(bA, bB, size, acc_nm, False)

    stage[pl.ds(0, L)] = (acc_m[0] + acc_m[1]) + (acc_m[2] + acc_m[3])
    stage[pl.ds(L, L)] = (acc_nm[0] + acc_nm[1]) + (acc_nm[2] + acc_nm[3])
    pltpu.sync_copy(stage, out.at[wid])


def _prep(idx, per_w, per_pad):
    biased = idx.astype(jnp.int32) + (jnp.arange(B, dtype=jnp.int32) * N_PIX)[:, None]
    flat = biased.reshape(-1)
    total = flat.shape[0]
    flat = jnp.pad(flat, (0, NW * per_w - total))
    return jnp.pad(flat.reshape(NW, per_w), ((0, 0), (0, per_pad - per_w))).reshape(-1)


def kernel(outA, outB, matchA, matchB, nonMatchA, nonMatchB, hardNegative):
    tableA = outA.reshape(B * N_PIX, D)
    tableB = outB.reshape(B * N_PIX, D)
    parts = _sc_loss(
        tableA, tableB,
        _prep(matchA, M_PER_W, M_PAD),
        _prep(matchB, M_PER_W, M_PAD),
        _prep(nonMatchA, NM_PER_W, NM_PAD),
        _prep(nonMatchB, NM_PER_W, NM_PAD),
    )
    matchLossSum = parts[:, :L].sum() / N_MATCH
    nonMatchLossSum = NONMATCH_W * parts[:, L:].sum() / N_NONMATCH
    contrastiveLossSum = matchLossSum + nonMatchLossSum
    return (contrastiveLossSum, matchLossSum, nonMatchLossSum)


# speedup vs baseline: 6.0130x; 1.0043x over previous
import functools

import jax
import jax.numpy as jnp
from jax import lax
from jax.experimental import pallas as pl
from jax.experimental.pallas import tpu as pltpu
from jax.experimental.pallas import tpu_sc as plsc

B = 2
N_PIX = 147456
D = 128
N_MATCH = 5000
N_NONMATCH = 20000
MARGIN = 0.5
NONMATCH_W = 1.0

NC = 2
NS = 16
NW = NC * NS
L = 16

M_TOT = B * N_MATCH
NM_TOT = B * N_NONMATCH
M_PER_W = -(-M_TOT // NW)
NM_PER_W = NM_TOT // NW

F_M = 320
F_NM = 1264
ROW_MAX = B * N_PIX - 1

T = 128


def _tiles(total):
    out, s = [], 0
    while s < total:
        out.append((s, min(T, total - s)))
        s += T
    return out


M_TILES = _tiles(M_PER_W)
NM_TILES = _tiles(NM_PER_W)

_mesh = plsc.VectorSubcoreMesh(core_axis_name="c", subcore_axis_name="s")


@functools.partial(
    pl.kernel,
    out_type=jax.ShapeDtypeStruct((NW, 2 * L), jnp.float32),
    mesh=_mesh,
    scratch_types=[
        pltpu.VMEM((F_M + 2 * L,), jnp.int32),
        pltpu.VMEM((F_M + 2 * L,), jnp.int32),
        pltpu.VMEM((F_NM + L,), jnp.int32),
        pltpu.VMEM((F_NM + L,), jnp.int32),
        pltpu.VMEM((F_M,), jnp.int32),
        pltpu.VMEM((F_M,), jnp.int32),
        pltpu.VMEM((F_NM,), jnp.int32),
        pltpu.VMEM((F_NM,), jnp.int32),
        pltpu.VMEM((T, D), jnp.float32),
        pltpu.VMEM((T, D), jnp.float32),
        pltpu.VMEM((T, D), jnp.float32),
        pltpu.VMEM((T, D), jnp.float32),
        pltpu.VMEM((2 * L,), jnp.float32),
        pltpu.SemaphoreType.DMA,
        pltpu.SemaphoreType.DMA,
        pltpu.SemaphoreType.DMA,
    ],
)
def _sc_loss(tableA, tableB, mA, mB, nmA, nmB, out,
             winMA, winMB, winNA, winNB,
             iAm_v, iBm_v, iAnm_v, iBnm_v,
             bufA0, bufB0, bufA1, bufB1, stage, sem0, sem1, sem_i):
    wid = lax.axis_index("s") * NC + lax.axis_index("c")

    base_m = wid * M_PER_W
    astart_m = jnp.minimum((base_m // 8) * 8, M_TOT - F_M)
    off_m = base_m - astart_m
    base_nm = wid * NM_PER_W
    astart_nm = jnp.minimum((base_nm // 8) * 8, NM_TOT - F_NM)
    off_nm = base_nm - astart_nm

    cps = [pltpu.async_copy(mA.at[pl.ds(astart_m, F_M)],
                            winMA.at[pl.ds(0, F_M)], sem_i),
           pltpu.async_copy(mB.at[pl.ds(astart_m, F_M)],
                            winMB.at[pl.ds(0, F_M)], sem_i),
           pltpu.async_copy(nmA.at[pl.ds(astart_nm, F_NM)],
                            winNA.at[pl.ds(0, F_NM)], sem_i),
           pltpu.async_copy(nmB.at[pl.ds(astart_nm, F_NM)],
                            winNB.at[pl.ds(0, F_NM)], sem_i)]
    for cp in cps:
        cp.wait()

    lanes = lax.iota(jnp.int32, L)

    def transform(n_chunks, base, off, boundary, wa, wb, da, db):
        @plsc.parallel_loop(0, n_chunks, 1)
        def _(k):
            j0 = k * L
            p = base + j0 + lanes
            bias = jnp.where(p >= boundary, jnp.int32(N_PIX), jnp.int32(0))
            ra = wa[pl.ds(off + j0, L)]
            rb = wb[pl.ds(off + j0, L)]
            da[pl.ds(j0, L)] = jnp.clip(ra + bias, 0, ROW_MAX)
            db[pl.ds(j0, L)] = jnp.clip(rb + bias, 0, ROW_MAX)

    m_valid = jnp.minimum(M_PER_W, M_TOT - base_m)

    bufs = [(bufA0, bufB0, sem0), (bufA1, bufB1, sem1)]
    tiles = [(True, s, z) for (s, z) in M_TILES] + \
            [(False, s, z) for (s, z) in NM_TILES]

    def issue(i):
        is_m, start, size = tiles[i]
        bA, bB, sem = bufs[i % 2]
        ia = (iAm_v if is_m else iAnm_v).at[pl.ds(start, size)]
        ib = (iBm_v if is_m else iBnm_v).at[pl.ds(start, size)]
        cpA = pltpu.async_copy(tableA.at[ia], bA.at[pl.ds(0, size)], sem)
        cpB = pltpu.async_copy(tableB.at[ib], bB.at[pl.ds(0, size)], sem)
        return cpA, cpB

    transform(F_M // L, base_m, off_m, N_MATCH, winMA, winMB, iAm_v, iBm_v)
    inflight = issue(0)
    transform(F_NM // L, base_nm, off_nm, N_NONMATCH, winNA, winNB,
              iAnm_v, iBnm_v)

    zero = jnp.zeros((L,), jnp.float32)

    def run_tile(bA, bB, n, accs, is_m):
        def body(r, acc):
            acc = list(acc)
            for j in range(D // L):
                a = bA[r, pl.ds(j * L, L)]
                b = bB[r, pl.ds(j * L, L)]
                d = a - b
                if is_m:
                    acc[j % 4] = acc[j % 4] + d * d
                else:
                    acc[j % 4] = acc[j % 4] + jnp.maximum(MARGIN - d * d, zero)
            return tuple(acc)
        return plsc.parallel_loop(0, n, 1, unroll=2, carry=accs)(body)

    acc_m = (zero, zero, zero, zero)
    acc_nm = (zero, zero, zero, zero)
    for i, (is_m, start, size) in enumerate(tiles):
        cpA, cpB = inflight
        if i + 1 < len(tiles):
            inflight = issue(i + 1)
        cpA.wait()
        cpB.wait()
        bA, bB, _ = bufs[i % 2]
        if is_m:
            n = jnp.clip(m_valid - start, 0, size)
            acc_m = run_tile(bA, bB, n, acc_m, True)
        else:
            acc_nm = run_tile(bA, bB, size, acc_nm, False)

    stage[pl.ds(0, L)] = (acc_m[0] + acc_m[1]) + (acc_m[2] + acc_m[3])
    stage[pl.ds(L, L)] = (acc_nm[0] + acc_nm[1]) + (acc_nm[2] + acc_nm[3])
    pltpu.sync_copy(stage, out.at[wid])


def kernel(outA, outB, matchA, matchB, nonMatchA, nonMatchB, hardNegative):
    i32 = jnp.int32
    parts = _sc_loss(
        outA.reshape(B * N_PIX, D),
        outB.reshape(B * N_PIX, D),
        matchA.astype(i32).reshape(M_TOT),
        matchB.astype(i32).reshape(M_TOT),
        nonMatchA.astype(i32).reshape(NM_TOT),
        nonMatchB.astype(i32).reshape(NM_TOT),
    )
    matchLossSum = parts[:, :L].sum() / N_MATCH
    nonMatchLossSum = NONMATCH_W * parts[:, L:].sum() / N_NONMATCH
    contrastiveLossSum = matchLossSum + nonMatchLossSum
    return (contrastiveLossSum, matchLossSum, nonMatchLossSum)
